# baseline (speedup 1.0000x reference)
import functools

import jax
import jax.numpy as jnp
from jax import lax
from jax.experimental import pallas as pl
from jax.experimental.pallas import tpu as pltpu
from jax.experimental.pallas import tpu_sc as plsc

_NC = 2
_NS = 16
_L = 16
_NW = _NC * _NS


def _sc_row_partials(x, centers, label):
    B, D = x.shape
    bpw = B // _NW
    half = bpw // 2
    nch = D // _L
    mesh = plsc.VectorSubcoreMesh(core_axis_name="c", subcore_axis_name="s")
    out_t = [jax.ShapeDtypeStruct((B * _L,), jnp.float32) for _ in range(2)]

    @functools.partial(
        pl.kernel,
        mesh=mesh,
        out_type=out_t,
        scratch_types=[
            pltpu.VMEM((bpw,), jnp.int32),
            pltpu.VMEM((bpw, D), jnp.float32),
            pltpu.VMEM((bpw, D), jnp.float32),
            pltpu.VMEM((bpw * _L,), jnp.float32),
            pltpu.VMEM((bpw * _L,), jnp.float32),
            pltpu.SemaphoreType.DMA,
            pltpu.SemaphoreType.DMA,
            pltpu.SemaphoreType.DMA,
        ],
    )
    def sc_kernel(x_hbm, centers_hbm, label_hbm, pxc_hbm, pcc_hbm,
                  idx_v, xv, cv, pxc_v, pcc_v, sem0, sem1, sem2):
        wid = lax.axis_index("s") * _NC + lax.axis_index("c")
        base = wid * bpw
        xcopy = pltpu.async_copy(x_hbm.at[pl.ds(base, bpw)], xv, sem2)
        pltpu.sync_copy(label_hbm.at[pl.ds(base, bpw)], idx_v)
        g0 = pltpu.async_copy(
            centers_hbm.at[idx_v.at[pl.ds(0, half)]], cv.at[pl.ds(0, half)],
            sem0)
        g1 = pltpu.async_copy(
            centers_hbm.at[idx_v.at[pl.ds(half, half)]],
            cv.at[pl.ds(half, half)], sem1)

        zero = jnp.zeros((_L,), jnp.float32)

        def make_loop(lo, hi):
            @plsc.parallel_loop(lo, hi, step=1, unroll=3)
            def rbody(r):
                axc, acc = zero, zero
                for j in range(nch):
                    xa = xv[r, pl.ds(j * _L, _L)]
                    ca = cv[r, pl.ds(j * _L, _L)]
                    axc = axc + xa * ca
                    acc = acc + ca * ca
                pxc_v[pl.ds(r * _L, _L)] = axc
                pcc_v[pl.ds(r * _L, _L)] = acc

        xcopy.wait()
        g0.wait()
        make_loop(0, half)
        g1.wait()
        make_loop(half, bpw)

        pltpu.sync_copy(pxc_v, pxc_hbm.at[pl.ds(base * _L, bpw * _L)])
        pltpu.sync_copy(pcc_v, pcc_hbm.at[pl.ds(base * _L, bpw * _L)])

    return sc_kernel(x, centers, label)


def _tc_xnorm(x):
    B, D = x.shape

    def body(x_ref, out_ref):
        xr = x_ref[...]
        out_ref[...] = jnp.sum(xr * xr, axis=2)

    return pl.pallas_call(
        body,
        out_shape=jax.ShapeDtypeStruct((B // 8, 8), jnp.float32),
    )(x.reshape(B // 8, 8, D))


def _tc_loss(xx, pxc, pcc, batch):
    def body(xx_ref, pxc_ref, pcc_ref, out_ref):
        lane = lax.broadcasted_iota(jnp.int32, (128, 8), 0)
        seg = lax.broadcasted_iota(jnp.int32, (128, 8), 1)
        sel = (lane // _L == seg).astype(jnp.float32)

        def rowsum(p):
            return jax.lax.dot(p, sel, precision=jax.lax.Precision.DEFAULT)

        xc = rowsum(pxc_ref[...])
        cc = rowsum(pcc_ref[...])
        xx = xx_ref[...]
        gx = jnp.maximum(jnp.sqrt(xx), 1e-12)
        gc = jnp.maximum(jnp.sqrt(cc), 1e-12)
        term = xx / (gx * gx) + cc / (gc * gc) - 2.0 * (xc / (gx * gc))
        out_ref[0, 0] = jnp.sum(term) / (2.0 * batch)

    n = batch * _L // 128
    out = pl.pallas_call(
        body,
        out_shape=jax.ShapeDtypeStruct((1, 1), jnp.float32),
        out_specs=pl.BlockSpec(memory_space=pltpu.SMEM),
    )(xx, pxc.reshape(n, 128), pcc.reshape(n, 128))
    return out[0, 0]


def kernel(x, centers, label):
    batch = x.shape[0]
    feat = x.reshape(batch, -1)
    xx = _tc_xnorm(feat)
    pxc, pcc = _sc_row_partials(feat, centers, label)
    return _tc_loss(xx, pxc, pcc, batch)


# speedup vs baseline: 1.0096x; 1.0096x over previous
import functools

import jax
import jax.numpy as jnp
from jax import lax
from jax.experimental import pallas as pl
from jax.experimental.pallas import tpu as pltpu
from jax.experimental.pallas import tpu_sc as plsc

_NC = 2
_NS = 16
_L = 16
_NW = _NC * _NS


def _sc_row_partials(x, centers, label):
    B, D = x.shape
    bpw = B // _NW
    half = bpw // 2
    nch = D // _L
    mesh = plsc.VectorSubcoreMesh(core_axis_name="c", subcore_axis_name="s")
    out_t = [jax.ShapeDtypeStruct((B * _L,), jnp.float32) for _ in range(2)]

    @functools.partial(
        pl.kernel,
        mesh=mesh,
        out_type=out_t,
        scratch_types=[
            pltpu.VMEM((bpw,), jnp.int32),
            pltpu.VMEM((bpw, D), jnp.float32),
            pltpu.VMEM((bpw, D), jnp.float32),
            pltpu.VMEM((bpw * _L,), jnp.float32),
            pltpu.VMEM((bpw * _L,), jnp.float32),
            pltpu.SemaphoreType.DMA,
            pltpu.SemaphoreType.DMA,
            pltpu.SemaphoreType.DMA,
        ],
    )
    def sc_kernel(x_hbm, centers_hbm, label_hbm, pxc_hbm, pcc_hbm,
                  idx_v, xv, cv, pxc_v, pcc_v, sem0, sem1, sem2):
        wid = lax.axis_index("s") * _NC + lax.axis_index("c")
        base = wid * bpw
        xcopy = pltpu.async_copy(x_hbm.at[pl.ds(base, bpw)], xv, sem2)
        pltpu.sync_copy(label_hbm.at[pl.ds(base, bpw)], idx_v)
        g0 = pltpu.async_copy(
            centers_hbm.at[idx_v.at[pl.ds(0, half)]], cv.at[pl.ds(0, half)],
            sem0)
        g1 = pltpu.async_copy(
            centers_hbm.at[idx_v.at[pl.ds(half, half)]],
            cv.at[pl.ds(half, half)], sem1)

        zero = jnp.zeros((_L,), jnp.float32)

        def make_loop(lo, hi):
            @plsc.parallel_loop(lo, hi, step=1, unroll=2)
            def rbody(r):
                axc, acc = zero, zero
                for j in range(nch):
                    xa = xv[r, pl.ds(j * _L, _L)]
                    ca = cv[r, pl.ds(j * _L, _L)]
                    axc = axc + xa * ca
                    acc = acc + ca * ca
                pxc_v[pl.ds(r * _L, _L)] = axc
                pcc_v[pl.ds(r * _L, _L)] = acc

        xcopy.wait()
        g0.wait()
        make_loop(0, half)
        g1.wait()
        make_loop(half, bpw)

        pltpu.sync_copy(pxc_v, pxc_hbm.at[pl.ds(base * _L, bpw * _L)])
        pltpu.sync_copy(pcc_v, pcc_hbm.at[pl.ds(base * _L, bpw * _L)])

    return sc_kernel(x, centers, label)


def _tc_xnorm(x):
    B, D = x.shape

    def body(x_ref, out_ref):
        xr = x_ref[...]
        out_ref[...] = jnp.sum(xr * xr, axis=2)

    return pl.pallas_call(
        body,
        out_shape=jax.ShapeDtypeStruct((B // 8, 8), jnp.float32),
    )(x.reshape(B // 8, 8, D))


def _tc_loss(xx, pxc, pcc, batch):
    def body(xx_ref, pxc_ref, pcc_ref, out_ref):
        lane = lax.broadcasted_iota(jnp.int32, (128, 8), 0)
        seg = lax.broadcasted_iota(jnp.int32, (128, 8), 1)
        sel = (lane // _L == seg).astype(jnp.float32)

        def rowsum(p):
            return jax.lax.dot(p, sel, precision=jax.lax.Precision.DEFAULT)

        xc = rowsum(pxc_ref[...])
        cc = rowsum(pcc_ref[...])
        xx = xx_ref[...]
        gx = jnp.maximum(jnp.sqrt(xx), 1e-12)
        gc = jnp.maximum(jnp.sqrt(cc), 1e-12)
        term = xx / (gx * gx) + cc / (gc * gc) - 2.0 * (xc / (gx * gc))
        out_ref[0, 0] = jnp.sum(term) / (2.0 * batch)

    n = batch * _L // 128
    out = pl.pallas_call(
        body,
        out_shape=jax.ShapeDtypeStruct((1, 1), jnp.float32),
        out_specs=pl.BlockSpec(memory_space=pltpu.SMEM),
    )(xx, pxc.reshape(n, 128), pcc.reshape(n, 128))
    return out[0, 0]


def kernel(x, centers, label):
    batch = x.shape[0]
    feat = x.reshape(batch, -1)
    xx = _tc_xnorm(feat)
    pxc, pcc = _sc_row_partials(feat, centers, label)
    return _tc_loss(xx, pxc, pcc, batch)
